# Initial kernel scaffold; baseline (speedup 1.0000x reference)
import jax
import jax.numpy as jnp
from jax.experimental import pallas as pl


def kernel(query, K, V, topk):
    raise NotImplementedError("write your pallas kernel here")


# speedup vs baseline: 4.2111x; 4.2111x over previous
import functools

import jax
import jax.numpy as jnp
from jax.experimental import pallas as pl
from jax.experimental.pallas import tpu as pltpu

TAU = 0.2
NB = 8192
SEG = 128
NSEL = 40
KMAX = 32
NEG = -1e30


def _pass1_kernel(n, q_ref, k_ref, logits_ref, segmax_ref):
    b = pl.program_id(0)
    q = q_ref[...]
    qn = q / jnp.clip(jnp.sqrt(jnp.sum(q * q, axis=1, keepdims=True)), 1e-12)
    kblk = k_ref[...]
    norm = jnp.sqrt(jnp.sum(kblk * kblk, axis=1, keepdims=True))
    kn = kblk / jnp.clip(norm, 1e-12)
    sim = jax.lax.dot_general(qn, kn, (((1,), (1,)), ((), ())),
                              preferred_element_type=jnp.float32) / TAU
    col = jax.lax.broadcasted_iota(jnp.int32, sim.shape, 1) + b * NB
    sim = jnp.where(col < n, sim, NEG)
    logits_ref[...] = sim
    s3 = sim.reshape(sim.shape[0], NB // SEG, SEG)
    segmax_ref[...] = jnp.max(s3, axis=2)[None]


def _pass2_kernel(nq, segmax_ref, logits_ref, v_ref, topk_ref, out_ref,
                  segids, segids_s, cand_ref, vals_ref, idxs_ref, idxs_s,
                  vsel, sem0, sem1, sem2):
    nseg = segmax_ref.shape[1]
    segmax = segmax_ref[...]
    lane = jax.lax.broadcasted_iota(jnp.int32, (nq, nseg), 1)
    for j in range(NSEL):
        m = jnp.max(segmax, axis=1, keepdims=True)
        pos = jnp.min(jnp.where(segmax == m, lane, jnp.int32(2**30)),
                      axis=1, keepdims=True)
        segids[:, j:j + 1] = pos
        segmax = jnp.where(lane == pos, NEG, segmax)

    cp = pltpu.make_async_copy(segids, segids_s, sem0)
    cp.start()
    cp.wait()

    def issue_seg(p, _):
        qq = p // NSEL
        jj = p - qq * NSEL
        s = segids_s[qq, jj]
        pltpu.make_async_copy(
            logits_ref.at[pl.ds(qq, 1), pl.ds(s * SEG, SEG)],
            cand_ref.at[pl.ds(qq, 1), pl.ds(jj * SEG, SEG)], sem1).start()
        return 0
    jax.lax.fori_loop(0, nq * NSEL, issue_seg, 0)

    def drain_seg(p, _):
        pltpu.make_async_copy(
            logits_ref.at[pl.ds(0, 1), pl.ds(0, SEG)],
            cand_ref.at[pl.ds(0, 1), pl.ds(0, SEG)], sem1).wait()
        return 0
    jax.lax.fori_loop(0, nq * NSEL, drain_seg, 0)

    cand = cand_ref[...]
    g3 = (segids[...][:, :, None] * SEG
          + jax.lax.broadcasted_iota(jnp.int32, (nq, NSEL, SEG), 2))
    gidx = g3.reshape(nq, NSEL * SEG)

    for r in range(KMAX):
        m = jnp.max(cand, axis=1, keepdims=True)
        gi = jnp.min(jnp.where(cand == m, gidx, jnp.int32(2**30)),
                     axis=1, keepdims=True)
        vals_ref[:, r:r + 1] = m
        idxs_ref[:, r:r + 1] = gi
        cand = jnp.where(gidx == gi, NEG, cand)

    vals = vals_ref[...]
    keep = jax.lax.broadcasted_iota(jnp.int32, (nq, KMAX), 1) < topk_ref[0]
    vals = jnp.where(keep, vals, -jnp.inf)
    m = jnp.max(vals, axis=1, keepdims=True)
    e = jnp.exp(vals - m)
    attn = e / jnp.sum(e, axis=1, keepdims=True)

    cp = pltpu.make_async_copy(idxs_ref, idxs_s, sem0)
    cp.start()
    cp.wait()

    def issue_v(p, _):
        qq = p // KMAX
        jj = p - qq * KMAX
        idx = idxs_s[qq, jj]
        pltpu.make_async_copy(v_ref.at[pl.ds(idx, 1), :],
                              vsel.at[pl.ds(p, 1), :], sem2).start()
        return 0
    jax.lax.fori_loop(0, nq * KMAX, issue_v, 0)

    def drain_v(p, _):
        pltpu.make_async_copy(v_ref.at[pl.ds(0, 1), :],
                              vsel.at[pl.ds(0, 1), :], sem2).wait()
        return 0
    jax.lax.fori_loop(0, nq * KMAX, drain_v, 0)

    vs = vsel[...].reshape(nq, KMAX, v_ref.shape[1])
    out_ref[...] = jnp.sum(attn[:, :, None] * vs, axis=1)


def kernel(query, K, V, topk):
    n, d = K.shape
    nq = query.shape[0]
    nblk = (n + NB - 1) // NB
    npad = nblk * NB
    nseg = npad // SEG

    logits, segmax = pl.pallas_call(
        functools.partial(_pass1_kernel, n),
        grid=(nblk,),
        in_specs=[
            pl.BlockSpec((nq, d), lambda b: (0, 0)),
            pl.BlockSpec((NB, d), lambda b: (b, 0)),
        ],
        out_specs=[
            pl.BlockSpec((nq, NB), lambda b: (0, b)),
            pl.BlockSpec((1, nq, NB // SEG), lambda b: (b, 0, 0)),
        ],
        out_shape=[
            jax.ShapeDtypeStruct((nq, npad), jnp.float32),
            jax.ShapeDtypeStruct((nblk, nq, NB // SEG), jnp.float32),
        ],
    )(query, K)
    segmax = segmax.transpose(1, 0, 2).reshape(nq, nseg)

    topk_arr = jnp.asarray(topk, jnp.int32).reshape(1)
    return pl.pallas_call(
        functools.partial(_pass2_kernel, nq),
        in_specs=[
            pl.BlockSpec((nq, nseg), lambda: (0, 0)),
            pl.BlockSpec(memory_space=pl.ANY),
            pl.BlockSpec(memory_space=pl.ANY),
            pl.BlockSpec(memory_space=pltpu.SMEM),
        ],
        out_specs=pl.BlockSpec((nq, d), lambda: (0, 0)),
        out_shape=jax.ShapeDtypeStruct((nq, d), jnp.float32),
        scratch_shapes=[
            pltpu.VMEM((nq, NSEL), jnp.int32),
            pltpu.SMEM((nq, NSEL), jnp.int32),
            pltpu.VMEM((nq, NSEL * SEG), jnp.float32),
            pltpu.VMEM((nq, KMAX), jnp.float32),
            pltpu.VMEM((nq, KMAX), jnp.int32),
            pltpu.SMEM((nq, KMAX), jnp.int32),
            pltpu.VMEM((nq * KMAX, d), jnp.float32),
            pltpu.SemaphoreType.DMA,
            pltpu.SemaphoreType.DMA,
            pltpu.SemaphoreType.DMA,
        ],
    )(segmax, logits, V, topk_arr)
